# baseline (speedup 1.0000x reference)
import functools

import jax
import jax.numpy as jnp
from jax import lax
from jax.experimental import pallas as pl
from jax.experimental.pallas import tpu as pltpu
from jax.experimental.pallas import tpu_sc as plsc

B, N, D = 16, 2048, 1024
NC, NS = 2, 16
NW = NC * NS
DW = D // 2
CH = 64
NCHUNK = N // CH
LANES = 16
JV = DW // LANES


def _body(x_hbm, out_hbm, buf, acc, sem0, sem1):
    wid = lax.axis_index("c") * NS + lax.axis_index("s")
    b = wid // 2
    c0 = (wid % 2) * DW

    sems = (sem0, sem1)
    zeros = jnp.zeros((LANES,), jnp.float32)
    for j in range(JV):
        acc[pl.ds(j * LANES, LANES)] = zeros

    def start(i, slot):
        pltpu.async_copy(
            x_hbm.at[b, pl.ds(i * CH, CH), pl.ds(c0, DW)],
            buf.at[slot], sems[slot])

    def wait(slot):
        pltpu.make_async_copy(
            x_hbm.at[b, pl.ds(0, CH), pl.ds(c0, DW)],
            buf.at[slot], sems[slot]).wait()

    def accumulate(slot):
        def row_body(r, carry):
            for j in range(JV):
                sl = pl.ds(j * LANES, LANES)
                plsc.addupdate(acc.at[sl], buf[slot, r, sl])
            return carry

        lax.fori_loop(0, CH, row_body, 0)

    start(0, 0)

    @pl.loop(0, NCHUNK, step=2)
    def _chunk(g):
        start(g + 1, 1)
        wait(0)
        accumulate(0)

        @pl.when(g + 2 < NCHUNK)
        def _():
            start(g + 2, 0)

        wait(1)
        accumulate(1)

    pltpu.sync_copy(acc, out_hbm.at[b, pl.ds(c0, DW)])


_mesh = plsc.VectorSubcoreMesh(core_axis_name="c", subcore_axis_name="s")

_sum_sc = functools.partial(
    pl.kernel,
    out_type=jax.ShapeDtypeStruct((B, D), jnp.float32),
    mesh=_mesh,
    scratch_types=[
        pltpu.VMEM((2, CH, DW), jnp.float32),
        pltpu.VMEM((DW,), jnp.float32),
        pltpu.SemaphoreType.DMA,
        pltpu.SemaphoreType.DMA,
    ],
)(_body)


@jax.jit
def kernel(inputs):
    return _sum_sc(inputs)


# speedup vs baseline: 3.3493x; 3.3493x over previous
import functools

import jax
import jax.numpy as jnp
from jax import lax
from jax.experimental import pallas as pl
from jax.experimental.pallas import tpu as pltpu
from jax.experimental.pallas import tpu_sc as plsc

B, N, D = 16, 2048, 1024
NC, NS = 2, 16
NW = NC * NS
DW = D // 2
CH = 64
NCHUNK = N // CH
LANES = 16
JV = DW // LANES


def _body(x_hbm, out_hbm, buf, acc, sem0, sem1):
    wid = lax.axis_index("c") * NS + lax.axis_index("s")
    b = wid // 2
    c0 = (wid % 2) * DW

    sems = (sem0, sem1)

    def start(i, slot):
        pltpu.async_copy(
            x_hbm.at[b, pl.ds(i * CH, CH), pl.ds(c0, DW)],
            buf.at[slot], sems[slot])

    def wait(slot):
        pltpu.make_async_copy(
            x_hbm.at[b, pl.ds(0, CH), pl.ds(c0, DW)],
            buf.at[slot], sems[slot]).wait()

    def accumulate(slot, carry):
        def row_body(r, carry):
            vals = [buf[slot, r, pl.ds(j * LANES, LANES)] for j in range(JV)]
            return tuple(c + v for c, v in zip(carry, vals))

        return lax.fori_loop(0, CH, row_body, carry)

    start(0, 0)
    zeros = tuple(jnp.zeros((LANES,), jnp.float32) for _ in range(JV))

    @pl.loop(0, NCHUNK, step=2, init_carry=zeros)
    def _chunk(g, carry):
        start(g + 1, 1)
        wait(0)
        carry = accumulate(0, carry)

        @pl.when(g + 2 < NCHUNK)
        def _():
            start(g + 2, 0)

        wait(1)
        return accumulate(1, carry)

    for j in range(JV):
        acc[pl.ds(j * LANES, LANES)] = _chunk[j]

    pltpu.sync_copy(acc, out_hbm.at[b, pl.ds(c0, DW)])


_mesh = plsc.VectorSubcoreMesh(core_axis_name="c", subcore_axis_name="s")

_sum_sc = functools.partial(
    pl.kernel,
    out_type=jax.ShapeDtypeStruct((B, D), jnp.float32),
    mesh=_mesh,
    scratch_types=[
        pltpu.VMEM((2, CH, DW), jnp.float32),
        pltpu.VMEM((DW,), jnp.float32),
        pltpu.SemaphoreType.DMA,
        pltpu.SemaphoreType.DMA,
    ],
)(_body)


@jax.jit
def kernel(inputs):
    return _sum_sc(inputs)
